# Initial kernel scaffold; baseline (speedup 1.0000x reference)
import jax
import jax.numpy as jnp
from jax.experimental import pallas as pl


def kernel(user_emb, item_emb, users, items, social_src, social_dst, ui_src, ui_dst):
    raise NotImplementedError("write your pallas kernel here")


# speedup vs baseline: 2.3813x; 2.3813x over previous
import functools

import jax
import jax.numpy as jnp
from jax import lax
from jax.experimental import pallas as pl
from jax.experimental.pallas import tpu as pltpu
from jax.experimental.pallas import tpu_sc as plsc

NC = 2
NS = 16
L = 16

D = 64
NQ = D // L
ROWCH = 8
GRP = 128 // L
FL = 128
RNG = 12544
GPAD = 128
RCH = 128
NFULL = RNG // RCH
EPS = 1e-8

_MESH = plsc.VectorSubcoreMesh(
    core_axis_name="c", subcore_axis_name="s", num_cores=NC, num_subcores=NS
)


def _splat(ref, i):
  return plsc.load_gather(ref, [jnp.full((L,), i, jnp.int32)])


def _iota16():
  return lax.iota(jnp.int32, L)


def _count_one(idx2d, cnt, idx_v, ones_v, c, s):
  er = idx2d.shape[0]
  half = er // NC
  trips = half // (NS * ROWCH)

  def trip(t, _):
    roff = c * half + (s * trips + t) * ROWCH
    pltpu.sync_copy(idx2d.at[pl.ds(roff, ROWCH), :], idx_v)
    for f in range(ROWCH):
      pltpu.sync_copy(ones_v, cnt.at[idx_v.at[f]], add=True)
    return 0

  lax.fori_loop(0, trips, trip, 0)


def _deg_body(ncnt_u, ncnt_i,
              ui_src, ui_dst, soc_dst, zeros1d,
              udeg, ideg, sdeg,
              ucnt, icnt, scnt, idx_v, ones_v, zv, cv):
  c = lax.axis_index("c")
  s = lax.axis_index("s")
  pltpu.sync_copy(zeros1d, zv)
  for j in range(ROWCH):
    ones_v[pl.ds(j * L, L)] = jnp.full((L,), 1.0, jnp.float32)

  for cnt, ncnt in ((ucnt, ncnt_u), (icnt, ncnt_i), (scnt, ncnt_u)):
    per_tile = ncnt // NS
    nz = per_tile // 1024
    def zbody(k, _, cnt=cnt, per_tile=per_tile):
      pltpu.sync_copy(zv, cnt.at[pl.ds(s * per_tile + k * 1024, 1024)])
      return 0
    lax.fori_loop(0, nz, zbody, 0)
  plsc.subcore_barrier()

  _count_one(ui_src, ucnt, idx_v, ones_v, c, s)
  _count_one(ui_dst, icnt, idx_v, ones_v, c, s)
  _count_one(soc_dst, scnt, idx_v, ones_v, c, s)
  plsc.subcore_barrier()

  for cnt, ncnt, out in ((ucnt, ncnt_u, udeg), (icnt, ncnt_i, ideg),
                         (scnt, ncnt_u, sdeg)):
    per_tile = ncnt // NS
    nz = per_tile // 1024
    def wbody(k, _, cnt=cnt, ncnt=ncnt, out=out, per_tile=per_tile):
      off = s * per_tile + k * 1024
      pltpu.sync_copy(cnt.at[pl.ds(off, 1024)], cv)
      pltpu.sync_copy(cv, out.at[pl.ds(c * ncnt + off, 1024)])
      return 0
    lax.fori_loop(0, nz, wbody, 0)


def _make_deg_kernel(er_ui, er_soc, ncnt_u, ncnt_i):
  return pl.kernel(
      functools.partial(_deg_body, ncnt_u, ncnt_i),
      out_type=(
          jax.ShapeDtypeStruct((NC * ncnt_u,), jnp.float32),
          jax.ShapeDtypeStruct((NC * ncnt_i,), jnp.float32),
          jax.ShapeDtypeStruct((NC * ncnt_u,), jnp.float32),
      ),
      mesh=_MESH,
      compiler_params=pltpu.CompilerParams(use_tc_tiling_on_sc=False,
                                           needs_layout_passes=False),
      scratch_types=[
          pltpu.MemorySpace.VMEM_SHARED((ncnt_u,), jnp.float32),
          pltpu.MemorySpace.VMEM_SHARED((ncnt_i,), jnp.float32),
          pltpu.MemorySpace.VMEM_SHARED((ncnt_u,), jnp.float32),
          pltpu.VMEM((ROWCH, 128), jnp.int32),
          pltpu.VMEM((128,), jnp.float32),
          pltpu.VMEM((1024,), jnp.float32),
          pltpu.VMEM((1024,), jnp.float32),
      ],
      name="gnn_degrees",
  )


def _agg_body(n_ranges, affine, ncnt,
              tbl, gidx, sidx, deg, base, zeros2d, out,
              accum, sidx_v, gidx_v, gbuf, lbuf, gflush, lflush,
              rows_v, acc_v, aux_v, d0_v, d1_v, out_v, zv):
  c = lax.axis_index("c")
  s = lax.axis_index("s")
  er = gidx.shape[0]
  trips = er // (NS * ROWCH)
  nrounds = n_ranges // NC
  pltpu.sync_copy(zeros2d, zv)

  def do_flush(fill):
    for j in range(FL // L):
      lane = _iota16() + j * L
      gflush[pl.ds(j * L, L)] = jnp.where(
          lane < fill, gbuf[pl.ds(j * L, L)], lane)
      lflush[pl.ds(j * L, L)] = jnp.where(
          lane < fill, lbuf[pl.ds(j * L, L)], RNG + lane)
    pltpu.sync_copy(tbl.at[gflush], rows_v)
    pltpu.sync_copy(rows_v, accum.at[lflush], add=True)

  def writeout_rows(lo, row0, nrows):
    pltpu.sync_copy(accum.at[pl.ds(row0, nrows), :],
                    acc_v.at[pl.ds(0, nrows), :])
    pltpu.sync_copy(deg.at[pl.ds(lo + row0, nrows)],
                    d0_v.at[pl.ds(0, nrows)])
    pltpu.sync_copy(deg.at[pl.ds(ncnt + lo + row0, nrows)],
                    d1_v.at[pl.ds(0, nrows)])
    if affine:
      pltpu.sync_copy(base.at[pl.ds(lo + row0, nrows), :],
                      aux_v.at[pl.ds(0, nrows), :])

    def rloop(i, _):
      dsp = _splat(d0_v, i) + _splat(d1_v, i)
      inv = 1.0 / jnp.maximum(dsp, 1.0)
      if affine:
        sw = 1.0 - dsp / (dsp + EPS)
      for q in range(NQ):
        av = acc_v[i, pl.ds(q * L, L)] * inv
        if affine:
          av = av + sw * aux_v[i, pl.ds(q * L, L)]
        out_v[i, pl.ds(q * L, L)] = av
      return 0

    lax.fori_loop(0, nrows, rloop, 0)
    pltpu.sync_copy(out_v.at[pl.ds(0, nrows), :],
                    out.at[pl.ds(lo + row0, nrows), :])

  for rnd in range(nrounds):
    rid = rnd * NC + c
    lo = rid * RNG

    nz = NFULL // NS + 1
    def zbody(kk, _):
      k = s + kk * NS
      @pl.when(k < NFULL)
      def _():
        pltpu.sync_copy(zv, accum.at[pl.ds(k * RCH, RCH), :])
      return 0
    lax.fori_loop(0, nz, zbody, 0)
    @pl.when(s == NS - 1)
    def _():
      pltpu.sync_copy(zv, accum.at[pl.ds(RNG, GPAD), :])
    plsc.subcore_barrier()

    def trip(t, fill):
      roff = (s * trips + t) * ROWCH
      pltpu.sync_copy(sidx.at[pl.ds(roff, ROWCH), :], sidx_v)
      pltpu.sync_copy(gidx.at[pl.ds(roff, ROWCH), :], gidx_v)

      def row(r, fill):
        for q in range(GRP):
          sv = sidx_v[r, pl.ds(q * L, L)]
          gv = gidx_v[r, pl.ds(q * L, L)]
          m = (sv >= lo) & (sv < lo + RNG)
          cum = plsc.cumsum(m.astype(jnp.int32))
          pos = fill + cum - 1
          plsc.store_scatter(lbuf, [pos], sv - lo, mask=m)
          plsc.store_scatter(gbuf, [pos], gv, mask=m)
          cnt = jnp.max(plsc.all_reduce_population_count(m))
          fill = fill + cnt

          @pl.when(fill >= FL)
          def _():
            for j in range(FL // L):
              gflush[pl.ds(j * L, L)] = gbuf[pl.ds(j * L, L)]
              lflush[pl.ds(j * L, L)] = lbuf[pl.ds(j * L, L)]
            pltpu.sync_copy(tbl.at[gflush], rows_v)
            pltpu.sync_copy(rows_v, accum.at[lflush], add=True)
            gbuf[pl.ds(0, L)] = gbuf[pl.ds(FL, L)]
            lbuf[pl.ds(0, L)] = lbuf[pl.ds(FL, L)]

          fill = jnp.where(fill >= FL, fill - FL, fill)
        return fill

      return lax.fori_loop(0, ROWCH, row, fill)

    fill = lax.fori_loop(0, trips, trip, jnp.int32(0))
    do_flush(fill)
    plsc.subcore_barrier()

    nw = NFULL // NS + 1
    def wchunk(kk, _):
      k = s + kk * NS
      @pl.when(k < NFULL)
      def _():
        writeout_rows(lo, k * RCH, RCH)
      return 0
    lax.fori_loop(0, nw, wchunk, 0)
    plsc.subcore_barrier()


def _make_agg_kernel(n_dst, er, ncnt, affine, name):
  n_ranges = n_dst // RNG
  body = functools.partial(_agg_body, n_ranges, affine, ncnt)
  if not affine:
    body2 = lambda tbl, gidx, sidx, deg, zeros2d, out, *sc: body(
        tbl, gidx, sidx, deg, None, zeros2d, out, *sc)
  else:
    body2 = body
  return pl.kernel(
      body2,
      out_type=jax.ShapeDtypeStruct((n_dst, D), jnp.float32),
      mesh=_MESH,
      compiler_params=pltpu.CompilerParams(use_tc_tiling_on_sc=False,
                                           needs_layout_passes=False),
      scratch_types=[
          pltpu.MemorySpace.VMEM_SHARED((RNG + GPAD, D), jnp.float32),
          pltpu.VMEM((ROWCH, 128), jnp.int32),
          pltpu.VMEM((ROWCH, 128), jnp.int32),
          pltpu.VMEM((FL + L,), jnp.int32),
          pltpu.VMEM((FL + L,), jnp.int32),
          pltpu.VMEM((FL,), jnp.int32),
          pltpu.VMEM((FL,), jnp.int32),
          pltpu.VMEM((FL, D), jnp.float32),
          pltpu.VMEM((RCH, D), jnp.float32),
          pltpu.VMEM((RCH, D), jnp.float32),
          pltpu.VMEM((RCH,), jnp.float32),
          pltpu.VMEM((RCH,), jnp.float32),
          pltpu.VMEM((RCH, D), jnp.float32),
          pltpu.VMEM((RCH, D), jnp.float32),
      ],
      name=name,
  )


def _final_body(u0, i0, soc1, ru1, soc2, ru2, users, items,
                pred, lu, li,
                uq, iq, r_u0, r_s1, r_r1, r_s2, r_r2, r_i0, fu_v, pr_v):
  c = lax.axis_index("c")
  s = lax.axis_index("s")
  wid = s * NC + c
  nq = users.shape[0]
  per_w = nq // (NC * NS)
  nch = per_w // RCH

  def chunk(k, _):
    q0 = wid * per_w + k * RCH
    pltpu.sync_copy(users.at[pl.ds(q0, RCH)], uq)
    pltpu.sync_copy(items.at[pl.ds(q0, RCH)], iq)
    pltpu.sync_copy(u0.at[uq], r_u0)
    pltpu.sync_copy(soc1.at[uq], r_s1)
    pltpu.sync_copy(ru1.at[uq], r_r1)
    pltpu.sync_copy(soc2.at[uq], r_s2)
    pltpu.sync_copy(ru2.at[uq], r_r2)
    pltpu.sync_copy(i0.at[iq], r_i0)

    def rloop(i, _):
      acc = jnp.full((L,), 0.0, jnp.float32)
      for q in range(NQ):
        sl = pl.ds(q * L, L)
        fu = (r_u0[i, sl]
              + 0.5 * (r_s1[i, sl] + r_r1[i, sl])
              + 0.5 * (r_s2[i, sl] + r_r2[i, sl]))
        fu_v[i, sl] = fu
        acc = acc + fu * r_i0[i, sl]
      tot = plsc.cumsum(acc)
      p = 1.0 / (1.0 + jnp.exp(-tot))
      plsc.store_scatter(pr_v, [jnp.full((L,), i, jnp.int32)], p,
                         mask=_iota16() == L - 1)
      return 0

    lax.fori_loop(0, RCH, rloop, 0)
    pltpu.sync_copy(fu_v, lu.at[pl.ds(q0, RCH), :])
    pltpu.sync_copy(r_i0, li.at[pl.ds(q0, RCH), :])
    pltpu.sync_copy(pr_v, pred.at[pl.ds(q0, RCH)])
    return 0

  lax.fori_loop(0, nch, chunk, 0)


def _make_final_kernel(nq):
  return pl.kernel(
      _final_body,
      out_type=(
          jax.ShapeDtypeStruct((nq,), jnp.float32),
          jax.ShapeDtypeStruct((nq, D), jnp.float32),
          jax.ShapeDtypeStruct((nq, D), jnp.float32),
      ),
      mesh=_MESH,
      compiler_params=pltpu.CompilerParams(use_tc_tiling_on_sc=False,
                                           needs_layout_passes=False),
      scratch_types=[
          pltpu.VMEM((RCH,), jnp.int32),
          pltpu.VMEM((RCH,), jnp.int32),
          pltpu.VMEM((RCH, D), jnp.float32),
          pltpu.VMEM((RCH, D), jnp.float32),
          pltpu.VMEM((RCH, D), jnp.float32),
          pltpu.VMEM((RCH, D), jnp.float32),
          pltpu.VMEM((RCH, D), jnp.float32),
          pltpu.VMEM((RCH, D), jnp.float32),
          pltpu.VMEM((RCH, D), jnp.float32),
          pltpu.VMEM((RCH,), jnp.float32),
      ],
      name="gnn_final",
  )


def _pad_idx(idx, n_garbage_base, multiple):
  e = idx.shape[0]
  e_pad = -(-e // multiple) * multiple
  if e_pad != e:
    pad = n_garbage_base + (jnp.arange(e_pad - e, dtype=jnp.int32) % GPAD)
    idx = jnp.concatenate([idx.astype(jnp.int32), pad])
  else:
    idx = idx.astype(jnp.int32)
  return idx.reshape(e_pad // 128, 128)


def kernel(user_emb, item_emb, users, items, social_src, social_dst,
           ui_src, ui_dst):
  U = user_emb.shape[0]
  I = item_emb.shape[0]
  users = users.astype(jnp.int32)
  items = items.astype(jnp.int32)

  upad = -(-U // RNG) * RNG
  ipad = -(-I // RNG) * RNG
  u0 = jnp.concatenate([user_emb, jnp.zeros((upad - U, D), jnp.float32)])
  i0 = jnp.concatenate([item_emb, jnp.zeros((ipad - I, D), jnp.float32)])

  mult = NC * NS * ROWCH * 128
  ui_src2d = _pad_idx(ui_src, U, mult)
  ui_dst2d = _pad_idx(ui_dst, I, mult)
  soc_src2d = _pad_idx(social_src, 0, mult)
  soc_dst2d = _pad_idx(social_dst, U, mult)

  ncnt_u = -(-upad // (NS * 1024)) * (NS * 1024)
  ncnt_i = -(-ipad // (NS * 1024)) * (NS * 1024)

  zeros1d = jnp.zeros((1024,), jnp.float32)
  zeros2d = jnp.zeros((RCH, D), jnp.float32)

  deg_k = _make_deg_kernel(ui_src2d.shape[0], soc_dst2d.shape[0],
                           ncnt_u, ncnt_i)
  udeg, ideg, sdeg = deg_k(ui_src2d, ui_dst2d, soc_dst2d, zeros1d)

  k_soc1 = _make_agg_kernel(upad, soc_src2d.shape[0], ncnt_u, False,
                            "gnn_soc1")
  soc1 = k_soc1(u0, soc_src2d, soc_dst2d, sdeg, zeros2d)

  k_ru1 = _make_agg_kernel(upad, ui_src2d.shape[0], ncnt_u, True, "gnn_ru1")
  ru1 = k_ru1(i0, ui_dst2d, ui_src2d, udeg, u0, zeros2d)

  k_ri1 = _make_agg_kernel(ipad, ui_src2d.shape[0], ncnt_i, True, "gnn_ri1")
  ri1 = k_ri1(u0, ui_src2d, ui_dst2d, ideg, i0, zeros2d)

  soc2 = _make_agg_kernel(upad, soc_src2d.shape[0], ncnt_u, False,
                          "gnn_soc2")(soc1, soc_src2d, soc_dst2d, sdeg,
                                      zeros2d)

  ru2 = _make_agg_kernel(upad, ui_src2d.shape[0], ncnt_u, True,
                         "gnn_ru2")(ri1, ui_dst2d, ui_src2d, udeg, ru1,
                                    zeros2d)

  pred, lu, li = _make_final_kernel(users.shape[0])(
      u0, i0, soc1, ru1, soc2, ru2, users, items)
  return pred, lu, li


# speedup vs baseline: 2.9396x; 1.2345x over previous
import functools

import jax
import jax.numpy as jnp
from jax import lax
from jax.experimental import pallas as pl
from jax.experimental.pallas import tpu as pltpu
from jax.experimental.pallas import tpu_sc as plsc

NC = 2
NS = 16
L = 16

D = 64
NQ = D // L
ROWCH = 4
GRP = 128 // L
FL = 128
RNG = 12544
GPAD = 128
RCH = 128
NFULL = RNG // RCH
EPS = 1e-8

_MESH = plsc.VectorSubcoreMesh(
    core_axis_name="c", subcore_axis_name="s", num_cores=NC, num_subcores=NS
)


def _splat(ref, i):
  return plsc.load_gather(ref, [jnp.full((L,), i, jnp.int32)])


def _iota16():
  return lax.iota(jnp.int32, L)


def _count_one(idx2d, cnt, idx_v, ones_v, semc, c, s):
  er = idx2d.shape[0]
  half = er // NC
  trips = half // (NS * ROWCH)

  def trip(t, _):
    roff = c * half + (s * trips + t) * ROWCH
    pltpu.sync_copy(idx2d.at[pl.ds(roff, ROWCH), :], idx_v)
    for f in range(ROWCH):
      pltpu.sync_copy(ones_v, cnt.at[idx_v.at[f]], add=True)
    return 0

  lax.fori_loop(0, trips, trip, 0)


def _deg_body(ncnt_u, ncnt_i,
              ui_src, ui_dst, soc_dst, zeros1d,
              udeg, ideg, sdeg,
              ucnt, icnt, scnt, idx_v, ones_v, zv, cv, semc):
  c = lax.axis_index("c")
  s = lax.axis_index("s")
  pltpu.sync_copy(zeros1d, zv)
  for j in range(128 // L):
    ones_v[pl.ds(j * L, L)] = jnp.full((L,), 1.0, jnp.float32)

  for cnt, ncnt in ((ucnt, ncnt_u), (icnt, ncnt_i), (scnt, ncnt_u)):
    per_tile = ncnt // NS
    nz = per_tile // 1024
    def zbody(k, _, cnt=cnt, per_tile=per_tile):
      pltpu.sync_copy(zv, cnt.at[pl.ds(s * per_tile + k * 1024, 1024)])
      return 0
    lax.fori_loop(0, nz, zbody, 0)
  plsc.subcore_barrier()

  _count_one(ui_src, ucnt, idx_v, ones_v, semc, c, s)
  _count_one(ui_dst, icnt, idx_v, ones_v, semc, c, s)
  _count_one(soc_dst, scnt, idx_v, ones_v, semc, c, s)
  plsc.subcore_barrier()

  for cnt, ncnt, out in ((ucnt, ncnt_u, udeg), (icnt, ncnt_i, ideg),
                         (scnt, ncnt_u, sdeg)):
    per_tile = ncnt // NS
    nz = per_tile // 1024
    def wbody(k, _, cnt=cnt, ncnt=ncnt, out=out, per_tile=per_tile):
      off = s * per_tile + k * 1024
      pltpu.sync_copy(cnt.at[pl.ds(off, 1024)], cv)
      pltpu.sync_copy(cv, out.at[pl.ds(c * ncnt + off, 1024)])
      return 0
    lax.fori_loop(0, nz, wbody, 0)


def _make_deg_kernel(er_ui, er_soc, ncnt_u, ncnt_i):
  return pl.kernel(
      functools.partial(_deg_body, ncnt_u, ncnt_i),
      out_type=(
          jax.ShapeDtypeStruct((NC * ncnt_u,), jnp.float32),
          jax.ShapeDtypeStruct((NC * ncnt_i,), jnp.float32),
          jax.ShapeDtypeStruct((NC * ncnt_u,), jnp.float32),
      ),
      mesh=_MESH,
      compiler_params=pltpu.CompilerParams(use_tc_tiling_on_sc=False,
                                           needs_layout_passes=False),
      scratch_types=[
          pltpu.MemorySpace.VMEM_SHARED((ncnt_u,), jnp.float32),
          pltpu.MemorySpace.VMEM_SHARED((ncnt_i,), jnp.float32),
          pltpu.MemorySpace.VMEM_SHARED((ncnt_u,), jnp.float32),
          pltpu.VMEM((ROWCH, 128), jnp.int32),
          pltpu.VMEM((128,), jnp.float32),
          pltpu.VMEM((1024,), jnp.float32),
          pltpu.VMEM((1024,), jnp.float32),
          pltpu.SemaphoreType.DMA,
      ],
      name="gnn_degrees",
  )


def _agg_body(n_ranges, affine, ncnt,
              tbl, gidx, sidx, deg, base, zeros2d, out,
              accum, sidx_v, gidx_v, gbuf2, lbuf2, rows3,
              acc_v, aux_v, d0_v, d1_v, semg, sems, semi):
  c = lax.axis_index("c")
  s = lax.axis_index("s")
  er = gidx.shape[0]
  trips = er // (NS * ROWCH)
  nrounds = n_ranges // NC

  def do_flush(fill):
    for j in range(FL // L):
      lane = _iota16() + j * L
      gbuf2[0, pl.ds(j * L, L)] = jnp.where(
          lane < fill, gbuf2[0, pl.ds(j * L, L)], lane)
      lbuf2[0, pl.ds(j * L, L)] = jnp.where(
          lane < fill, lbuf2[0, pl.ds(j * L, L)], RNG + lane)
    pltpu.sync_copy(tbl.at[gbuf2.at[0]], rows3.at[0])
    pltpu.sync_copy(rows3.at[0], accum.at[lbuf2.at[0]], add=True)

  def writeout_rows(lo, row0, nrows):
    pltpu.sync_copy(accum.at[pl.ds(row0, nrows), :],
                    acc_v.at[pl.ds(0, nrows), :])
    pltpu.sync_copy(deg.at[pl.ds(lo + row0, nrows)],
                    d0_v.at[pl.ds(0, nrows)])
    pltpu.sync_copy(deg.at[pl.ds(ncnt + lo + row0, nrows)],
                    d1_v.at[pl.ds(0, nrows)])
    if affine:
      pltpu.sync_copy(base.at[pl.ds(lo + row0, nrows), :],
                      aux_v.at[pl.ds(0, nrows), :])

    def rloop(i, _):
      dsp = _splat(d0_v, i) + _splat(d1_v, i)
      inv = 1.0 / jnp.maximum(dsp, 1.0)
      if affine:
        sw = 1.0 - dsp / (dsp + EPS)
      for q in range(NQ):
        av = acc_v[i, pl.ds(q * L, L)] * inv
        if affine:
          av = av + sw * aux_v[i, pl.ds(q * L, L)]
        acc_v[i, pl.ds(q * L, L)] = av
      return 0

    lax.fori_loop(0, nrows, rloop, 0)
    pltpu.sync_copy(acc_v.at[pl.ds(0, nrows), :],
                    out.at[pl.ds(lo + row0, nrows), :])

  for rnd in range(nrounds):
    rid = rnd * NC + c
    lo = rid * RNG

    pltpu.sync_copy(zeros2d, aux_v)
    nz = NFULL // NS + 1
    def zbody(kk, _):
      k = s + kk * NS
      @pl.when(k < NFULL)
      def _():
        pltpu.sync_copy(aux_v, accum.at[pl.ds(k * RCH, RCH), :])
      return 0
    lax.fori_loop(0, nz, zbody, 0)
    @pl.when(s == NS - 1)
    def _():
      pltpu.sync_copy(aux_v, accum.at[pl.ds(RNG, GPAD), :])
    plsc.subcore_barrier()

    def trip(t, fill):
      roff = (s * trips + t) * ROWCH
      pltpu.sync_copy(sidx.at[pl.ds(roff, ROWCH), :], sidx_v)
      pltpu.sync_copy(gidx.at[pl.ds(roff, ROWCH), :], gidx_v)

      def row(r, fill):
        for q in range(GRP):
          sv = sidx_v[r, pl.ds(q * L, L)]
          gv = gidx_v[r, pl.ds(q * L, L)]
          m = (sv >= lo) & (sv < lo + RNG)
          cum = plsc.cumsum(m.astype(jnp.int32))
          pos = fill + cum - 1
          prow = lax.shift_right_logical(pos, 7)
          pcol = lax.bitwise_and(pos, 127)
          plsc.store_scatter(lbuf2, [prow, pcol], sv - lo, mask=m)
          plsc.store_scatter(gbuf2, [prow, pcol], gv, mask=m)
          cnt = jnp.max(plsc.all_reduce_population_count(m))
          fill = fill + cnt
        return fill

      fill = lax.fori_loop(0, ROWCH, row, fill)

      nf = lax.shift_right_logical(fill, 7)
      for b in range(ROWCH):
        @pl.when(b < nf)
        def _(b=b):
          pltpu.async_copy(tbl.at[gbuf2.at[b]], rows3.at[b], semg)
      for b in range(ROWCH):
        @pl.when(b < nf)
        def _(b=b):
          pltpu.make_async_copy(tbl.at[gbuf2.at[b]], rows3.at[b],
                                semg).wait()
      for b in range(ROWCH):
        @pl.when(b < nf)
        def _(b=b):
          pltpu.sync_copy(rows3.at[b], accum.at[lbuf2.at[b]], add=True)

      for b in range(1, ROWCH + 1):
        @pl.when(nf == b)
        def _(b=b):
          for j in range(FL // L):
            gv2 = gbuf2[b, pl.ds(j * L, L)]
            lv2 = lbuf2[b, pl.ds(j * L, L)]
            gbuf2[0, pl.ds(j * L, L)] = gv2
            lbuf2[0, pl.ds(j * L, L)] = lv2
      return lax.bitwise_and(fill, 127)

    fill = lax.fori_loop(0, trips, trip, jnp.int32(0))
    do_flush(fill)
    plsc.subcore_barrier()

    nw = NFULL // NS + 1
    def wchunk(kk, _):
      k = s + kk * NS
      @pl.when(k < NFULL)
      def _():
        writeout_rows(lo, k * RCH, RCH)
      return 0
    lax.fori_loop(0, nw, wchunk, 0)
    plsc.subcore_barrier()


def _make_agg_kernel(n_dst, er, ncnt, affine, name):
  n_ranges = n_dst // RNG
  body = functools.partial(_agg_body, n_ranges, affine, ncnt)
  if not affine:
    body2 = lambda tbl, gidx, sidx, deg, zeros2d, out, *sc: body(
        tbl, gidx, sidx, deg, None, zeros2d, out, *sc)
  else:
    body2 = body
  return pl.kernel(
      body2,
      out_type=jax.ShapeDtypeStruct((n_dst, D), jnp.float32),
      mesh=_MESH,
      compiler_params=pltpu.CompilerParams(use_tc_tiling_on_sc=False,
                                           needs_layout_passes=False),
      scratch_types=[
          pltpu.MemorySpace.VMEM_SHARED((RNG + GPAD, D), jnp.float32),
          pltpu.VMEM((ROWCH, 128), jnp.int32),
          pltpu.VMEM((ROWCH, 128), jnp.int32),
          pltpu.VMEM((ROWCH + 1, FL), jnp.int32),
          pltpu.VMEM((ROWCH + 1, FL), jnp.int32),
          pltpu.VMEM((ROWCH, FL, D), jnp.float32),
          pltpu.VMEM((RCH, D), jnp.float32),
          pltpu.VMEM((RCH, D), jnp.float32),
          pltpu.VMEM((RCH,), jnp.float32),
          pltpu.VMEM((RCH,), jnp.float32),
          pltpu.SemaphoreType.DMA,
          pltpu.SemaphoreType.DMA,
          pltpu.SemaphoreType.DMA,
      ],
      name=name,
  )


def _final_body(u0, i0, soc1, ru1, soc2, ru2, users, items,
                pred, lu, li,
                uq, iq, r_u0, r_s1, r_r1, r_s2, r_r2, r_i0, fu_v, pr_v):
  c = lax.axis_index("c")
  s = lax.axis_index("s")
  wid = s * NC + c
  nq = users.shape[0]
  per_w = nq // (NC * NS)
  nch = per_w // RCH

  def chunk(k, _):
    q0 = wid * per_w + k * RCH
    pltpu.sync_copy(users.at[pl.ds(q0, RCH)], uq)
    pltpu.sync_copy(items.at[pl.ds(q0, RCH)], iq)
    pltpu.sync_copy(u0.at[uq], r_u0)
    pltpu.sync_copy(soc1.at[uq], r_s1)
    pltpu.sync_copy(ru1.at[uq], r_r1)
    pltpu.sync_copy(soc2.at[uq], r_s2)
    pltpu.sync_copy(ru2.at[uq], r_r2)
    pltpu.sync_copy(i0.at[iq], r_i0)

    def rloop(i, _):
      acc = jnp.full((L,), 0.0, jnp.float32)
      for q in range(NQ):
        sl = pl.ds(q * L, L)
        fu = (r_u0[i, sl]
              + 0.5 * (r_s1[i, sl] + r_r1[i, sl])
              + 0.5 * (r_s2[i, sl] + r_r2[i, sl]))
        fu_v[i, sl] = fu
        acc = acc + fu * r_i0[i, sl]
      tot = plsc.cumsum(acc)
      p = 1.0 / (1.0 + jnp.exp(-tot))
      plsc.store_scatter(pr_v, [jnp.full((L,), i, jnp.int32)], p,
                         mask=_iota16() == L - 1)
      return 0

    lax.fori_loop(0, RCH, rloop, 0)
    pltpu.sync_copy(fu_v, lu.at[pl.ds(q0, RCH), :])
    pltpu.sync_copy(r_i0, li.at[pl.ds(q0, RCH), :])
    pltpu.sync_copy(pr_v, pred.at[pl.ds(q0, RCH)])
    return 0

  lax.fori_loop(0, nch, chunk, 0)


def _make_final_kernel(nq):
  return pl.kernel(
      _final_body,
      out_type=(
          jax.ShapeDtypeStruct((nq,), jnp.float32),
          jax.ShapeDtypeStruct((nq, D), jnp.float32),
          jax.ShapeDtypeStruct((nq, D), jnp.float32),
      ),
      mesh=_MESH,
      compiler_params=pltpu.CompilerParams(use_tc_tiling_on_sc=False,
                                           needs_layout_passes=False),
      scratch_types=[
          pltpu.VMEM((RCH,), jnp.int32),
          pltpu.VMEM((RCH,), jnp.int32),
          pltpu.VMEM((RCH, D), jnp.float32),
          pltpu.VMEM((RCH, D), jnp.float32),
          pltpu.VMEM((RCH, D), jnp.float32),
          pltpu.VMEM((RCH, D), jnp.float32),
          pltpu.VMEM((RCH, D), jnp.float32),
          pltpu.VMEM((RCH, D), jnp.float32),
          pltpu.VMEM((RCH, D), jnp.float32),
          pltpu.VMEM((RCH,), jnp.float32),
      ],
      name="gnn_final",
  )


def _pad_idx(idx, n_garbage_base, multiple):
  e = idx.shape[0]
  e_pad = -(-e // multiple) * multiple
  if e_pad != e:
    pad = n_garbage_base + (jnp.arange(e_pad - e, dtype=jnp.int32) % GPAD)
    idx = jnp.concatenate([idx.astype(jnp.int32), pad])
  else:
    idx = idx.astype(jnp.int32)
  return idx.reshape(e_pad // 128, 128)


def kernel(user_emb, item_emb, users, items, social_src, social_dst,
           ui_src, ui_dst):
  U = user_emb.shape[0]
  I = item_emb.shape[0]
  users = users.astype(jnp.int32)
  items = items.astype(jnp.int32)

  upad = -(-U // RNG) * RNG
  ipad = -(-I // RNG) * RNG
  u0 = jnp.concatenate([user_emb, jnp.zeros((upad - U, D), jnp.float32)])
  i0 = jnp.concatenate([item_emb, jnp.zeros((ipad - I, D), jnp.float32)])

  mult = NC * NS * ROWCH * 128
  ui_src2d = _pad_idx(ui_src, U, mult)
  ui_dst2d = _pad_idx(ui_dst, I, mult)
  soc_src2d = _pad_idx(social_src, 0, mult)
  soc_dst2d = _pad_idx(social_dst, U, mult)

  ncnt_u = -(-upad // (NS * 1024)) * (NS * 1024)
  ncnt_i = -(-ipad // (NS * 1024)) * (NS * 1024)

  zeros1d = jnp.zeros((1024,), jnp.float32)
  zeros2d = jnp.zeros((RCH, D), jnp.float32)

  deg_k = _make_deg_kernel(ui_src2d.shape[0], soc_dst2d.shape[0],
                           ncnt_u, ncnt_i)
  udeg, ideg, sdeg = deg_k(ui_src2d, ui_dst2d, soc_dst2d, zeros1d)

  k_soc1 = _make_agg_kernel(upad, soc_src2d.shape[0], ncnt_u, False,
                            "gnn_soc1")
  soc1 = k_soc1(u0, soc_src2d, soc_dst2d, sdeg, zeros2d)

  k_ru1 = _make_agg_kernel(upad, ui_src2d.shape[0], ncnt_u, True, "gnn_ru1")
  ru1 = k_ru1(i0, ui_dst2d, ui_src2d, udeg, u0, zeros2d)

  k_ri1 = _make_agg_kernel(ipad, ui_src2d.shape[0], ncnt_i, True, "gnn_ri1")
  ri1 = k_ri1(u0, ui_src2d, ui_dst2d, ideg, i0, zeros2d)

  soc2 = _make_agg_kernel(upad, soc_src2d.shape[0], ncnt_u, False,
                          "gnn_soc2")(soc1, soc_src2d, soc_dst2d, sdeg,
                                      zeros2d)

  ru2 = _make_agg_kernel(upad, ui_src2d.shape[0], ncnt_u, True,
                         "gnn_ru2")(ri1, ui_dst2d, ui_src2d, udeg, ru1,
                                    zeros2d)

  pred, lu, li = _make_final_kernel(users.shape[0])(
      u0, i0, soc1, ru1, soc2, ru2, users, items)
  return pred, lu, li


# speedup vs baseline: 4.3140x; 1.4676x over previous
import functools

import jax
import jax.numpy as jnp
from jax import lax
from jax.experimental import pallas as pl
from jax.experimental.pallas import tpu as pltpu
from jax.experimental.pallas import tpu_sc as plsc

NC = 2
NS = 16
L = 16

D = 64
NQ = D // L
ROWCH = 4
GRP = 128 // L
FL = 128
RNG = 12544
GPAD = 128
RCH = 128
NFULL = RNG // RCH
EPS = 1e-8

_MESH = plsc.VectorSubcoreMesh(
    core_axis_name="c", subcore_axis_name="s", num_cores=NC, num_subcores=NS
)


def _splat(ref, i):
  return plsc.load_gather(ref, [jnp.full((L,), i, jnp.int32)])


def _iota16():
  return lax.iota(jnp.int32, L)


def _count_one(idx2d, cnt, idx_v, ones_v, semc, c, s):
  er = idx2d.shape[0]
  half = er // NC
  trips = half // (NS * ROWCH)

  def trip(t, _):
    roff = c * half + (s * trips + t) * ROWCH
    pltpu.sync_copy(idx2d.at[pl.ds(roff, ROWCH), :], idx_v)
    for f in range(ROWCH):
      pltpu.async_copy(ones_v, cnt.at[idx_v.at[f]], semc, add=True)
    for f in range(ROWCH):
      pltpu.make_async_copy(ones_v, cnt.at[idx_v.at[f]], semc).wait()
    return 0

  lax.fori_loop(0, trips, trip, 0)


def _deg_body(ncnt_u, ncnt_i,
              ui_src, ui_dst, soc_dst, zeros1d,
              udeg, ideg, sdeg,
              ucnt, icnt, scnt, idx_v, ones_v, zv, cv, semc):
  c = lax.axis_index("c")
  s = lax.axis_index("s")
  pltpu.sync_copy(zeros1d, zv)
  for j in range(128 // L):
    ones_v[pl.ds(j * L, L)] = jnp.full((L,), 1.0, jnp.float32)

  for cnt, ncnt in ((ucnt, ncnt_u), (icnt, ncnt_i), (scnt, ncnt_u)):
    per_tile = ncnt // NS
    nz = per_tile // 1024
    def zbody(k, _, cnt=cnt, per_tile=per_tile):
      pltpu.sync_copy(zv, cnt.at[pl.ds(s * per_tile + k * 1024, 1024)])
      return 0
    lax.fori_loop(0, nz, zbody, 0)
  plsc.subcore_barrier()

  _count_one(ui_src, ucnt, idx_v, ones_v, semc, c, s)
  _count_one(ui_dst, icnt, idx_v, ones_v, semc, c, s)
  _count_one(soc_dst, scnt, idx_v, ones_v, semc, c, s)
  plsc.subcore_barrier()

  for cnt, ncnt, out in ((ucnt, ncnt_u, udeg), (icnt, ncnt_i, ideg),
                         (scnt, ncnt_u, sdeg)):
    per_tile = ncnt // NS
    nz = per_tile // 1024
    def wbody(k, _, cnt=cnt, ncnt=ncnt, out=out, per_tile=per_tile):
      off = s * per_tile + k * 1024
      pltpu.sync_copy(cnt.at[pl.ds(off, 1024)], cv)
      pltpu.sync_copy(cv, out.at[pl.ds(c * ncnt + off, 1024)])
      return 0
    lax.fori_loop(0, nz, wbody, 0)


def _make_deg_kernel(er_ui, er_soc, ncnt_u, ncnt_i):
  return pl.kernel(
      functools.partial(_deg_body, ncnt_u, ncnt_i),
      out_type=(
          jax.ShapeDtypeStruct((NC * ncnt_u,), jnp.float32),
          jax.ShapeDtypeStruct((NC * ncnt_i,), jnp.float32),
          jax.ShapeDtypeStruct((NC * ncnt_u,), jnp.float32),
      ),
      mesh=_MESH,
      compiler_params=pltpu.CompilerParams(use_tc_tiling_on_sc=False,
                                           needs_layout_passes=False),
      scratch_types=[
          pltpu.MemorySpace.VMEM_SHARED((ncnt_u,), jnp.float32),
          pltpu.MemorySpace.VMEM_SHARED((ncnt_i,), jnp.float32),
          pltpu.MemorySpace.VMEM_SHARED((ncnt_u,), jnp.float32),
          pltpu.VMEM((ROWCH, 128), jnp.int32),
          pltpu.VMEM((128,), jnp.float32),
          pltpu.VMEM((1024,), jnp.float32),
          pltpu.VMEM((1024,), jnp.float32),
          pltpu.SemaphoreType.DMA,
      ],
      name="gnn_degrees",
  )


def _agg_body(n_ranges, affine, ncnt,
              tbl, gidx, sidx, deg, base, zeros2d, out,
              accum, sidx_v, gidx_v, gbuf2, lbuf2, rows3,
              acc_v, aux_v, d0_v, d1_v, semg, sems, semi):
  c = lax.axis_index("c")
  s = lax.axis_index("s")
  er = gidx.shape[0]
  trips = er // (NS * ROWCH)
  nrounds = n_ranges // NC

  def do_flush(fill):
    for j in range(FL // L):
      lane = _iota16() + j * L
      gbuf2[0, pl.ds(j * L, L)] = jnp.where(
          lane < fill, gbuf2[0, pl.ds(j * L, L)], lane)
      lbuf2[0, pl.ds(j * L, L)] = jnp.where(
          lane < fill, lbuf2[0, pl.ds(j * L, L)], RNG + lane)
    pltpu.sync_copy(tbl.at[gbuf2.at[0]], rows3.at[0])
    pltpu.sync_copy(rows3.at[0], accum.at[lbuf2.at[0]], add=True)

  def writeout_rows(lo, row0, nrows):
    pltpu.sync_copy(accum.at[pl.ds(row0, nrows), :],
                    acc_v.at[pl.ds(0, nrows), :])
    pltpu.sync_copy(deg.at[pl.ds(lo + row0, nrows)],
                    d0_v.at[pl.ds(0, nrows)])
    pltpu.sync_copy(deg.at[pl.ds(ncnt + lo + row0, nrows)],
                    d1_v.at[pl.ds(0, nrows)])
    if affine:
      pltpu.sync_copy(base.at[pl.ds(lo + row0, nrows), :],
                      aux_v.at[pl.ds(0, nrows), :])

    def rloop(i, _):
      dsp = _splat(d0_v, i) + _splat(d1_v, i)
      inv = 1.0 / jnp.maximum(dsp, 1.0)
      if affine:
        sw = 1.0 - dsp / (dsp + EPS)
      for q in range(NQ):
        av = acc_v[i, pl.ds(q * L, L)] * inv
        if affine:
          av = av + sw * aux_v[i, pl.ds(q * L, L)]
        acc_v[i, pl.ds(q * L, L)] = av
      return 0

    lax.fori_loop(0, nrows, rloop, 0)
    pltpu.sync_copy(acc_v.at[pl.ds(0, nrows), :],
                    out.at[pl.ds(lo + row0, nrows), :])

  for rnd in range(nrounds):
    rid = rnd * NC + c
    lo = rid * RNG

    pltpu.sync_copy(zeros2d, aux_v)
    nz = NFULL // NS + 1
    def zbody(kk, _):
      k = s + kk * NS
      @pl.when(k < NFULL)
      def _():
        pltpu.sync_copy(aux_v, accum.at[pl.ds(k * RCH, RCH), :])
      return 0
    lax.fori_loop(0, nz, zbody, 0)
    @pl.when(s == NS - 1)
    def _():
      pltpu.sync_copy(aux_v, accum.at[pl.ds(RNG, GPAD), :])
    plsc.subcore_barrier()

    def stage(t, pp):
      roff = (s * trips + t) * ROWCH
      pltpu.async_copy(sidx.at[pl.ds(roff, ROWCH), :], sidx_v.at[pp], semi)
      pltpu.async_copy(gidx.at[pl.ds(roff, ROWCH), :], gidx_v.at[pp], semi)

    def unstage(t, pp):
      roff = (s * trips + t) * ROWCH
      pltpu.make_async_copy(sidx.at[pl.ds(roff, ROWCH), :], sidx_v.at[pp],
                            semi).wait()
      pltpu.make_async_copy(gidx.at[pl.ds(roff, ROWCH), :], gidx_v.at[pp],
                            semi).wait()

    stage(0, 0)

    def trip(t, fill):
      p = lax.bitwise_and(t, 1)
      for pp in range(2):
        @pl.when(p == pp)
        def _(pp=pp):
          unstage(t, pp)
          @pl.when(t + 1 < trips)
          def _():
            stage(t + 1, 1 - pp)

      def row(r, fill):
        for q in range(GRP):
          sv = jnp.where(
              p == 0,
              sidx_v[0, r, pl.ds(q * L, L)],
              sidx_v[1, r, pl.ds(q * L, L)])
          gv = jnp.where(
              p == 0,
              gidx_v[0, r, pl.ds(q * L, L)],
              gidx_v[1, r, pl.ds(q * L, L)])
          m = (sv >= lo) & (sv < lo + RNG)
          cum = plsc.cumsum(m.astype(jnp.int32))
          pos = fill + cum - 1
          prow = lax.shift_right_logical(pos, 7)
          pcol = lax.bitwise_and(pos, 127)
          plsc.store_scatter(lbuf2, [prow, pcol], sv - lo, mask=m)
          plsc.store_scatter(gbuf2, [prow, pcol], gv, mask=m)
          cnt = jnp.max(plsc.all_reduce_population_count(m))
          fill = fill + cnt
        return fill

      fill = lax.fori_loop(0, ROWCH, row, fill)

      nf = lax.shift_right_logical(fill, 7)
      for b in range(ROWCH):
        @pl.when(b < nf)
        def _(b=b):
          pltpu.async_copy(tbl.at[gbuf2.at[b]], rows3.at[b], semg)
      for b in range(ROWCH):
        @pl.when(b < nf)
        def _(b=b):
          pltpu.make_async_copy(tbl.at[gbuf2.at[b]], rows3.at[b],
                                semg).wait()
      for b in range(ROWCH):
        @pl.when(b < nf)
        def _(b=b):
          pltpu.sync_copy(rows3.at[b], accum.at[lbuf2.at[b]], add=True)

      for b in range(1, ROWCH + 1):
        @pl.when(nf == b)
        def _(b=b):
          for j in range(FL // L):
            gv2 = gbuf2[b, pl.ds(j * L, L)]
            lv2 = lbuf2[b, pl.ds(j * L, L)]
            gbuf2[0, pl.ds(j * L, L)] = gv2
            lbuf2[0, pl.ds(j * L, L)] = lv2
      return lax.bitwise_and(fill, 127)

    fill = lax.fori_loop(0, trips, trip, jnp.int32(0))
    do_flush(fill)
    plsc.subcore_barrier()

    nw = NFULL // NS + 1
    def wchunk(kk, _):
      k = s + kk * NS
      @pl.when(k < NFULL)
      def _():
        writeout_rows(lo, k * RCH, RCH)
      return 0
    lax.fori_loop(0, nw, wchunk, 0)
    plsc.subcore_barrier()


def _make_agg_kernel(n_dst, er, ncnt, affine, name):
  n_ranges = n_dst // RNG
  body = functools.partial(_agg_body, n_ranges, affine, ncnt)
  if not affine:
    body2 = lambda tbl, gidx, sidx, deg, zeros2d, out, *sc: body(
        tbl, gidx, sidx, deg, None, zeros2d, out, *sc)
  else:
    body2 = body
  return pl.kernel(
      body2,
      out_type=jax.ShapeDtypeStruct((n_dst, D), jnp.float32),
      mesh=_MESH,
      compiler_params=pltpu.CompilerParams(use_tc_tiling_on_sc=False,
                                           needs_layout_passes=False),
      scratch_types=[
          pltpu.MemorySpace.VMEM_SHARED((RNG + GPAD, D), jnp.float32),
          pltpu.VMEM((2, ROWCH, 128), jnp.int32),
          pltpu.VMEM((2, ROWCH, 128), jnp.int32),
          pltpu.VMEM((ROWCH + 1, FL), jnp.int32),
          pltpu.VMEM((ROWCH + 1, FL), jnp.int32),
          pltpu.VMEM((ROWCH, FL, D), jnp.float32),
          pltpu.VMEM((RCH, D), jnp.float32),
          pltpu.VMEM((RCH, D), jnp.float32),
          pltpu.VMEM((RCH,), jnp.float32),
          pltpu.VMEM((RCH,), jnp.float32),
          pltpu.SemaphoreType.DMA,
          pltpu.SemaphoreType.DMA,
          pltpu.SemaphoreType.DMA,
      ],
      name=name,
  )


def _final_body(u0, i0, soc1, ru1, soc2, ru2, users, items,
                pred, lu, li,
                uq, iq, r_u0, r_s1, r_r1, r_s2, r_r2, r_i0, fu_v, pr_v):
  c = lax.axis_index("c")
  s = lax.axis_index("s")
  wid = s * NC + c
  nq = users.shape[0]
  per_w = nq // (NC * NS)
  nch = per_w // RCH

  def chunk(k, _):
    q0 = wid * per_w + k * RCH
    pltpu.sync_copy(users.at[pl.ds(q0, RCH)], uq)
    pltpu.sync_copy(items.at[pl.ds(q0, RCH)], iq)
    pltpu.sync_copy(u0.at[uq], r_u0)
    pltpu.sync_copy(soc1.at[uq], r_s1)
    pltpu.sync_copy(ru1.at[uq], r_r1)
    pltpu.sync_copy(soc2.at[uq], r_s2)
    pltpu.sync_copy(ru2.at[uq], r_r2)
    pltpu.sync_copy(i0.at[iq], r_i0)

    def rloop(i, _):
      acc = jnp.full((L,), 0.0, jnp.float32)
      for q in range(NQ):
        sl = pl.ds(q * L, L)
        fu = (r_u0[i, sl]
              + 0.5 * (r_s1[i, sl] + r_r1[i, sl])
              + 0.5 * (r_s2[i, sl] + r_r2[i, sl]))
        fu_v[i, sl] = fu
        acc = acc + fu * r_i0[i, sl]
      tot = plsc.cumsum(acc)
      p = 1.0 / (1.0 + jnp.exp(-tot))
      plsc.store_scatter(pr_v, [jnp.full((L,), i, jnp.int32)], p,
                         mask=_iota16() == L - 1)
      return 0

    lax.fori_loop(0, RCH, rloop, 0)
    pltpu.sync_copy(fu_v, lu.at[pl.ds(q0, RCH), :])
    pltpu.sync_copy(r_i0, li.at[pl.ds(q0, RCH), :])
    pltpu.sync_copy(pr_v, pred.at[pl.ds(q0, RCH)])
    return 0

  lax.fori_loop(0, nch, chunk, 0)


def _make_final_kernel(nq):
  return pl.kernel(
      _final_body,
      out_type=(
          jax.ShapeDtypeStruct((nq,), jnp.float32),
          jax.ShapeDtypeStruct((nq, D), jnp.float32),
          jax.ShapeDtypeStruct((nq, D), jnp.float32),
      ),
      mesh=_MESH,
      compiler_params=pltpu.CompilerParams(use_tc_tiling_on_sc=False,
                                           needs_layout_passes=False),
      scratch_types=[
          pltpu.VMEM((RCH,), jnp.int32),
          pltpu.VMEM((RCH,), jnp.int32),
          pltpu.VMEM((RCH, D), jnp.float32),
          pltpu.VMEM((RCH, D), jnp.float32),
          pltpu.VMEM((RCH, D), jnp.float32),
          pltpu.VMEM((RCH, D), jnp.float32),
          pltpu.VMEM((RCH, D), jnp.float32),
          pltpu.VMEM((RCH, D), jnp.float32),
          pltpu.VMEM((RCH, D), jnp.float32),
          pltpu.VMEM((RCH,), jnp.float32),
      ],
      name="gnn_final",
  )


def _pad_idx(idx, n_garbage_base, multiple):
  e = idx.shape[0]
  e_pad = -(-e // multiple) * multiple
  if e_pad != e:
    pad = n_garbage_base + (jnp.arange(e_pad - e, dtype=jnp.int32) % GPAD)
    idx = jnp.concatenate([idx.astype(jnp.int32), pad])
  else:
    idx = idx.astype(jnp.int32)
  return idx.reshape(e_pad // 128, 128)


def kernel(user_emb, item_emb, users, items, social_src, social_dst,
           ui_src, ui_dst):
  U = user_emb.shape[0]
  I = item_emb.shape[0]
  users = users.astype(jnp.int32)
  items = items.astype(jnp.int32)

  upad = -(-U // RNG) * RNG
  ipad = -(-I // RNG) * RNG
  u0 = jnp.concatenate([user_emb, jnp.zeros((upad - U, D), jnp.float32)])
  i0 = jnp.concatenate([item_emb, jnp.zeros((ipad - I, D), jnp.float32)])

  mult = NC * NS * ROWCH * 128
  ui_src2d = _pad_idx(ui_src, U, mult)
  ui_dst2d = _pad_idx(ui_dst, I, mult)
  soc_src2d = _pad_idx(social_src, 0, mult)
  soc_dst2d = _pad_idx(social_dst, U, mult)

  ncnt_u = -(-upad // (NS * 1024)) * (NS * 1024)
  ncnt_i = -(-ipad // (NS * 1024)) * (NS * 1024)

  zeros1d = jnp.zeros((1024,), jnp.float32)
  zeros2d = jnp.zeros((RCH, D), jnp.float32)

  deg_k = _make_deg_kernel(ui_src2d.shape[0], soc_dst2d.shape[0],
                           ncnt_u, ncnt_i)
  udeg, ideg, sdeg = deg_k(ui_src2d, ui_dst2d, soc_dst2d, zeros1d)

  k_soc1 = _make_agg_kernel(upad, soc_src2d.shape[0], ncnt_u, False,
                            "gnn_soc1")
  soc1 = k_soc1(u0, soc_src2d, soc_dst2d, sdeg, zeros2d)

  k_ru1 = _make_agg_kernel(upad, ui_src2d.shape[0], ncnt_u, True, "gnn_ru1")
  ru1 = k_ru1(i0, ui_dst2d, ui_src2d, udeg, u0, zeros2d)

  k_ri1 = _make_agg_kernel(ipad, ui_src2d.shape[0], ncnt_i, True, "gnn_ri1")
  ri1 = k_ri1(u0, ui_src2d, ui_dst2d, ideg, i0, zeros2d)

  soc2 = _make_agg_kernel(upad, soc_src2d.shape[0], ncnt_u, False,
                          "gnn_soc2")(soc1, soc_src2d, soc_dst2d, sdeg,
                                      zeros2d)

  ru2 = _make_agg_kernel(upad, ui_src2d.shape[0], ncnt_u, True,
                         "gnn_ru2")(ri1, ui_dst2d, ui_src2d, udeg, ru1,
                                    zeros2d)

  pred, lu, li = _make_final_kernel(users.shape[0])(
      u0, i0, soc1, ru1, soc2, ru2, users, items)
  return pred, lu, li


# speedup vs baseline: 4.6776x; 1.0843x over previous
import functools

import jax
import jax.numpy as jnp
from jax import lax
from jax.experimental import pallas as pl
from jax.experimental.pallas import tpu as pltpu
from jax.experimental.pallas import tpu_sc as plsc

NC = 2
NS = 16
L = 16

D = 64
NQ = D // L
ROWCH = 4
GRP = 128 // L
FL = 128
RNG = 12544
RNG_U = 16768
GPAD = 128
RCH = 128
NFULL = RNG // RCH
EPS = 1e-8

_MESH = plsc.VectorSubcoreMesh(
    core_axis_name="c", subcore_axis_name="s", num_cores=NC, num_subcores=NS
)


def _splat(ref, i):
  return plsc.load_gather(ref, [jnp.full((L,), i, jnp.int32)])


def _iota16():
  return lax.iota(jnp.int32, L)


def _count_one(idx2d, cnt, idx_v, ones_v, semc, c, s):
  er = idx2d.shape[0]
  half = er // NC
  trips = half // (NS * ROWCH)

  def trip(t, _):
    roff = c * half + (s * trips + t) * ROWCH
    pltpu.sync_copy(idx2d.at[pl.ds(roff, ROWCH), :], idx_v)
    for f in range(ROWCH):
      pltpu.async_copy(ones_v, cnt.at[idx_v.at[f]], semc, add=True)
    for f in range(ROWCH):
      pltpu.make_async_copy(ones_v, cnt.at[idx_v.at[f]], semc).wait()
    return 0

  lax.fori_loop(0, trips, trip, 0)


def _deg_body(ncnt_u, ncnt_i,
              ui_src, ui_dst, soc_dst, zeros1d,
              udeg, ideg, sdeg,
              ucnt, icnt, scnt, idx_v, ones_v, zv, cv, semc):
  c = lax.axis_index("c")
  s = lax.axis_index("s")
  pltpu.sync_copy(zeros1d, zv)
  for j in range(128 // L):
    ones_v[pl.ds(j * L, L)] = jnp.full((L,), 1.0, jnp.float32)

  for cnt, ncnt in ((ucnt, ncnt_u), (icnt, ncnt_i), (scnt, ncnt_u)):
    per_tile = ncnt // NS
    nz = per_tile // 1024
    def zbody(k, _, cnt=cnt, per_tile=per_tile):
      pltpu.sync_copy(zv, cnt.at[pl.ds(s * per_tile + k * 1024, 1024)])
      return 0
    lax.fori_loop(0, nz, zbody, 0)
  plsc.subcore_barrier()

  _count_one(ui_src, ucnt, idx_v, ones_v, semc, c, s)
  _count_one(ui_dst, icnt, idx_v, ones_v, semc, c, s)
  _count_one(soc_dst, scnt, idx_v, ones_v, semc, c, s)
  plsc.subcore_barrier()

  for cnt, ncnt, out in ((ucnt, ncnt_u, udeg), (icnt, ncnt_i, ideg),
                         (scnt, ncnt_u, sdeg)):
    per_tile = ncnt // NS
    nz = per_tile // 1024
    def wbody(k, _, cnt=cnt, ncnt=ncnt, out=out, per_tile=per_tile):
      off = s * per_tile + k * 1024
      pltpu.sync_copy(cnt.at[pl.ds(off, 1024)], cv)
      pltpu.sync_copy(cv, out.at[pl.ds(c * ncnt + off, 1024)])
      return 0
    lax.fori_loop(0, nz, wbody, 0)


def _make_deg_kernel(er_ui, er_soc, ncnt_u, ncnt_i):
  return pl.kernel(
      functools.partial(_deg_body, ncnt_u, ncnt_i),
      out_type=(
          jax.ShapeDtypeStruct((NC * ncnt_u,), jnp.float32),
          jax.ShapeDtypeStruct((NC * ncnt_i,), jnp.float32),
          jax.ShapeDtypeStruct((NC * ncnt_u,), jnp.float32),
      ),
      mesh=_MESH,
      compiler_params=pltpu.CompilerParams(use_tc_tiling_on_sc=False,
                                           needs_layout_passes=False),
      scratch_types=[
          pltpu.MemorySpace.VMEM_SHARED((ncnt_u,), jnp.float32),
          pltpu.MemorySpace.VMEM_SHARED((ncnt_i,), jnp.float32),
          pltpu.MemorySpace.VMEM_SHARED((ncnt_u,), jnp.float32),
          pltpu.VMEM((ROWCH, 128), jnp.int32),
          pltpu.VMEM((128,), jnp.float32),
          pltpu.VMEM((1024,), jnp.float32),
          pltpu.VMEM((1024,), jnp.float32),
          pltpu.SemaphoreType.DMA,
      ],
      name="gnn_degrees",
  )


def _agg_body(n_ranges, affine, ncnt, rng,
              tbl, gidx, sidx, deg, base, zeros2d, out,
              accum, sidx_v, gidx_v, gbuf2, lbuf2, rows3,
              acc_v, aux_v, d0_v, d1_v, semg, sems, semi):
  c = lax.axis_index("c")
  s = lax.axis_index("s")
  er = gidx.shape[0]
  trips = er // (NS * ROWCH)
  nrounds = n_ranges // NC

  def do_flush(fill):
    for j in range(FL // L):
      lane = _iota16() + j * L
      gbuf2[0, pl.ds(j * L, L)] = jnp.where(
          lane < fill, gbuf2[0, pl.ds(j * L, L)], lane)
      lbuf2[0, pl.ds(j * L, L)] = jnp.where(
          lane < fill, lbuf2[0, pl.ds(j * L, L)], rng + lane)
    pltpu.sync_copy(tbl.at[gbuf2.at[0]], rows3.at[0])
    pltpu.sync_copy(rows3.at[0], accum.at[lbuf2.at[0]], add=True)

  def writeout_rows(lo, row0, nrows):
    pltpu.sync_copy(accum.at[pl.ds(row0, nrows), :],
                    acc_v.at[pl.ds(0, nrows), :])
    pltpu.sync_copy(deg.at[pl.ds(lo + row0, nrows)],
                    d0_v.at[pl.ds(0, nrows)])
    pltpu.sync_copy(deg.at[pl.ds(ncnt + lo + row0, nrows)],
                    d1_v.at[pl.ds(0, nrows)])
    if affine:
      pltpu.sync_copy(base.at[pl.ds(lo + row0, nrows), :],
                      aux_v.at[pl.ds(0, nrows), :])

    def rloop(i, _):
      dsp = _splat(d0_v, i) + _splat(d1_v, i)
      inv = 1.0 / jnp.maximum(dsp, 1.0)
      if affine:
        sw = 1.0 - dsp / (dsp + EPS)
      for q in range(NQ):
        av = acc_v[i, pl.ds(q * L, L)] * inv
        if affine:
          av = av + sw * aux_v[i, pl.ds(q * L, L)]
        acc_v[i, pl.ds(q * L, L)] = av
      return 0

    lax.fori_loop(0, nrows, rloop, 0)
    pltpu.sync_copy(acc_v.at[pl.ds(0, nrows), :],
                    out.at[pl.ds(lo + row0, nrows), :])

  for rnd in range(nrounds):
    rid = rnd * NC + c
    lo = rid * rng
    nfull = rng // RCH

    pltpu.sync_copy(zeros2d, aux_v)
    nz = nfull // NS + 1
    def zbody(kk, _):
      k = s + kk * NS
      @pl.when(k < nfull)
      def _():
        pltpu.sync_copy(aux_v, accum.at[pl.ds(k * RCH, RCH), :])
      return 0
    lax.fori_loop(0, nz, zbody, 0)
    @pl.when(s == NS - 1)
    def _():
      pltpu.sync_copy(aux_v, accum.at[pl.ds(rng, GPAD), :])
    plsc.subcore_barrier()

    def stage(t, pp):
      roff = (s * trips + t) * ROWCH
      pltpu.async_copy(sidx.at[pl.ds(roff, ROWCH), :], sidx_v.at[pp], semi)
      pltpu.async_copy(gidx.at[pl.ds(roff, ROWCH), :], gidx_v.at[pp], semi)

    def unstage(t, pp):
      roff = (s * trips + t) * ROWCH
      pltpu.make_async_copy(sidx.at[pl.ds(roff, ROWCH), :], sidx_v.at[pp],
                            semi).wait()
      pltpu.make_async_copy(gidx.at[pl.ds(roff, ROWCH), :], gidx_v.at[pp],
                            semi).wait()

    stage(0, 0)

    def trip(t, fill):
      p = lax.bitwise_and(t, 1)
      for pp in range(2):
        @pl.when(p == pp)
        def _(pp=pp):
          unstage(t, pp)
          @pl.when(t + 1 < trips)
          def _():
            stage(t + 1, 1 - pp)

      def row(r, fill):
        for q in range(GRP):
          sv = jnp.where(
              p == 0,
              sidx_v[0, r, pl.ds(q * L, L)],
              sidx_v[1, r, pl.ds(q * L, L)])
          gv = jnp.where(
              p == 0,
              gidx_v[0, r, pl.ds(q * L, L)],
              gidx_v[1, r, pl.ds(q * L, L)])
          m = (sv >= lo) & (sv < lo + rng)
          cum = plsc.cumsum(m.astype(jnp.int32))
          pos = fill + cum - 1
          prow = lax.shift_right_logical(pos, 7)
          pcol = lax.bitwise_and(pos, 127)
          plsc.store_scatter(lbuf2, [prow, pcol], sv - lo, mask=m)
          plsc.store_scatter(gbuf2, [prow, pcol], gv, mask=m)
          cnt = jnp.max(plsc.all_reduce_population_count(m))
          fill = fill + cnt
        return fill

      fill = lax.fori_loop(0, ROWCH, row, fill)

      nf = lax.shift_right_logical(fill, 7)
      for b in range(ROWCH):
        @pl.when(b < nf)
        def _(b=b):
          pltpu.async_copy(tbl.at[gbuf2.at[b]], rows3.at[b], semg)
      for b in range(ROWCH):
        @pl.when(b < nf)
        def _(b=b):
          pltpu.make_async_copy(tbl.at[gbuf2.at[b]], rows3.at[b],
                                semg).wait()
      for b in range(ROWCH):
        @pl.when(b < nf)
        def _(b=b):
          pltpu.sync_copy(rows3.at[b], accum.at[lbuf2.at[b]], add=True)

      for b in range(1, ROWCH + 1):
        @pl.when(nf == b)
        def _(b=b):
          for j in range(FL // L):
            gv2 = gbuf2[b, pl.ds(j * L, L)]
            lv2 = lbuf2[b, pl.ds(j * L, L)]
            gbuf2[0, pl.ds(j * L, L)] = gv2
            lbuf2[0, pl.ds(j * L, L)] = lv2
      return lax.bitwise_and(fill, 127)

    fill = lax.fori_loop(0, trips, trip, jnp.int32(0))
    do_flush(fill)
    plsc.subcore_barrier()

    nw = nfull // NS + 1
    def wchunk(kk, _):
      k = s + kk * NS
      @pl.when(k < nfull)
      def _():
        writeout_rows(lo, k * RCH, RCH)
      return 0
    lax.fori_loop(0, nw, wchunk, 0)
    plsc.subcore_barrier()


def _make_agg_kernel(n_dst, er, ncnt, affine, name, rng=RNG):
  n_ranges = n_dst // rng
  body = functools.partial(_agg_body, n_ranges, affine, ncnt, rng)
  if not affine:
    body2 = lambda tbl, gidx, sidx, deg, zeros2d, out, *sc: body(
        tbl, gidx, sidx, deg, None, zeros2d, out, *sc)
  else:
    body2 = body
  return pl.kernel(
      body2,
      out_type=jax.ShapeDtypeStruct((n_dst, D), jnp.float32),
      mesh=_MESH,
      compiler_params=pltpu.CompilerParams(use_tc_tiling_on_sc=False,
                                           needs_layout_passes=False),
      scratch_types=[
          pltpu.MemorySpace.VMEM_SHARED((rng + GPAD, D), jnp.float32),
          pltpu.VMEM((2, ROWCH, 128), jnp.int32),
          pltpu.VMEM((2, ROWCH, 128), jnp.int32),
          pltpu.VMEM((ROWCH + 1, FL), jnp.int32),
          pltpu.VMEM((ROWCH + 1, FL), jnp.int32),
          pltpu.VMEM((ROWCH, FL, D), jnp.float32),
          pltpu.VMEM((RCH, D), jnp.float32),
          pltpu.VMEM((RCH, D), jnp.float32),
          pltpu.VMEM((RCH,), jnp.float32),
          pltpu.VMEM((RCH,), jnp.float32),
          pltpu.SemaphoreType.DMA,
          pltpu.SemaphoreType.DMA,
          pltpu.SemaphoreType.DMA,
      ],
      name=name,
  )


def _final_body(u0, i0, soc1, ru1, soc2, ru2, users, items,
                pred, lu, li,
                uq, iq, r_u0, r_s1, r_r1, r_s2, r_r2, r_i0, fu_v, pr_v):
  c = lax.axis_index("c")
  s = lax.axis_index("s")
  wid = s * NC + c
  nq = users.shape[0]
  per_w = nq // (NC * NS)
  nch = per_w // RCH

  def chunk(k, _):
    q0 = wid * per_w + k * RCH
    pltpu.sync_copy(users.at[pl.ds(q0, RCH)], uq)
    pltpu.sync_copy(items.at[pl.ds(q0, RCH)], iq)
    pltpu.sync_copy(u0.at[uq], r_u0)
    pltpu.sync_copy(soc1.at[uq], r_s1)
    pltpu.sync_copy(ru1.at[uq], r_r1)
    pltpu.sync_copy(soc2.at[uq], r_s2)
    pltpu.sync_copy(ru2.at[uq], r_r2)
    pltpu.sync_copy(i0.at[iq], r_i0)

    def rloop(i, _):
      acc = jnp.full((L,), 0.0, jnp.float32)
      for q in range(NQ):
        sl = pl.ds(q * L, L)
        fu = (r_u0[i, sl]
              + 0.5 * (r_s1[i, sl] + r_r1[i, sl])
              + 0.5 * (r_s2[i, sl] + r_r2[i, sl]))
        fu_v[i, sl] = fu
        acc = acc + fu * r_i0[i, sl]
      tot = plsc.cumsum(acc)
      p = 1.0 / (1.0 + jnp.exp(-tot))
      plsc.store_scatter(pr_v, [jnp.full((L,), i, jnp.int32)], p,
                         mask=_iota16() == L - 1)
      return 0

    lax.fori_loop(0, RCH, rloop, 0)
    pltpu.sync_copy(fu_v, lu.at[pl.ds(q0, RCH), :])
    pltpu.sync_copy(r_i0, li.at[pl.ds(q0, RCH), :])
    pltpu.sync_copy(pr_v, pred.at[pl.ds(q0, RCH)])
    return 0

  lax.fori_loop(0, nch, chunk, 0)


def _make_final_kernel(nq):
  return pl.kernel(
      _final_body,
      out_type=(
          jax.ShapeDtypeStruct((nq,), jnp.float32),
          jax.ShapeDtypeStruct((nq, D), jnp.float32),
          jax.ShapeDtypeStruct((nq, D), jnp.float32),
      ),
      mesh=_MESH,
      compiler_params=pltpu.CompilerParams(use_tc_tiling_on_sc=False,
                                           needs_layout_passes=False),
      scratch_types=[
          pltpu.VMEM((RCH,), jnp.int32),
          pltpu.VMEM((RCH,), jnp.int32),
          pltpu.VMEM((RCH, D), jnp.float32),
          pltpu.VMEM((RCH, D), jnp.float32),
          pltpu.VMEM((RCH, D), jnp.float32),
          pltpu.VMEM((RCH, D), jnp.float32),
          pltpu.VMEM((RCH, D), jnp.float32),
          pltpu.VMEM((RCH, D), jnp.float32),
          pltpu.VMEM((RCH, D), jnp.float32),
          pltpu.VMEM((RCH,), jnp.float32),
      ],
      name="gnn_final",
  )


def _pad_idx(idx, n_garbage_base, multiple):
  e = idx.shape[0]
  e_pad = -(-e // multiple) * multiple
  if e_pad != e:
    pad = n_garbage_base + (jnp.arange(e_pad - e, dtype=jnp.int32) % GPAD)
    idx = jnp.concatenate([idx.astype(jnp.int32), pad])
  else:
    idx = idx.astype(jnp.int32)
  return idx.reshape(e_pad // 128, 128)


def kernel(user_emb, item_emb, users, items, social_src, social_dst,
           ui_src, ui_dst):
  U = user_emb.shape[0]
  I = item_emb.shape[0]
  users = users.astype(jnp.int32)
  items = items.astype(jnp.int32)

  upad = -(-U // RNG_U) * RNG_U
  ipad = -(-I // RNG) * RNG
  u0 = jnp.concatenate([user_emb, jnp.zeros((upad - U, D), jnp.float32)])
  i0 = jnp.concatenate([item_emb, jnp.zeros((ipad - I, D), jnp.float32)])

  mult = NC * NS * ROWCH * 128
  ui_src2d = _pad_idx(ui_src, U, mult)
  ui_dst2d = _pad_idx(ui_dst, I, mult)
  soc_src2d = _pad_idx(social_src, 0, mult)
  soc_dst2d = _pad_idx(social_dst, U, mult)

  ncnt_u = -(-upad // (NS * 1024)) * (NS * 1024)
  ncnt_i = -(-ipad // (NS * 1024)) * (NS * 1024)

  zeros1d = jnp.zeros((1024,), jnp.float32)
  zeros2d = jnp.zeros((RCH, D), jnp.float32)

  deg_k = _make_deg_kernel(ui_src2d.shape[0], soc_dst2d.shape[0],
                           ncnt_u, ncnt_i)
  udeg, ideg, sdeg = deg_k(ui_src2d, ui_dst2d, soc_dst2d, zeros1d)

  k_soc1 = _make_agg_kernel(upad, soc_src2d.shape[0], ncnt_u, False,
                            "gnn_soc1", rng=RNG_U)
  soc1 = k_soc1(u0, soc_src2d, soc_dst2d, sdeg, zeros2d)

  k_ru1 = _make_agg_kernel(upad, ui_src2d.shape[0], ncnt_u, True,
                           "gnn_ru1", rng=RNG_U)
  ru1 = k_ru1(i0, ui_dst2d, ui_src2d, udeg, u0, zeros2d)

  k_ri1 = _make_agg_kernel(ipad, ui_src2d.shape[0], ncnt_i, True, "gnn_ri1")
  ri1 = k_ri1(u0, ui_src2d, ui_dst2d, ideg, i0, zeros2d)

  soc2 = _make_agg_kernel(upad, soc_src2d.shape[0], ncnt_u, False,
                          "gnn_soc2", rng=RNG_U)(soc1, soc_src2d, soc_dst2d, sdeg,
                                      zeros2d)

  ru2 = _make_agg_kernel(upad, ui_src2d.shape[0], ncnt_u, True,
                         "gnn_ru2", rng=RNG_U)(ri1, ui_dst2d, ui_src2d, udeg, ru1,
                                    zeros2d)

  pred, lu, li = _make_final_kernel(users.shape[0])(
      u0, i0, soc1, ru1, soc2, ru2, users, items)
  return pred, lu, li
